# baseline (speedup 1.0000x reference)
import jax
import jax.numpy as jnp
from jax.experimental import pallas as pl
from jax.experimental.pallas import tpu as pltpu

_BM = 400


def _gcn_kernel(a_ref, ax_ref, w0t_ref, b0_ref, a0_ref, w1t_ref, b1_ref,
                a1_ref, out_ref, t_ref):
    @pl.when(pl.program_id(0) == 0)
    def _compute_t():
        x = jnp.dot(ax_ref[...], w0t_ref[...], preferred_element_type=jnp.float32)
        x = x + b0_ref[...]
        a0 = a0_ref[0, 0]
        x = jnp.where(x >= 0, x, a0 * x)
        t = jnp.dot(x, w1t_ref[...], preferred_element_type=jnp.float32)
        t_ref[...] = t + b1_ref[...]

    acc = jnp.dot(a_ref[...], t_ref[...], preferred_element_type=jnp.float32)
    a1 = a1_ref[0, 0]
    out_ref[...] = jnp.where(acc >= 0, acc, a1 * acc)


def kernel(A, AX, W0, b0, a0, W1, b1, a1):
    n, d = AX.shape
    h = W0.shape[0]

    out = pl.pallas_call(
        _gcn_kernel,
        grid=(n // _BM,),
        in_specs=[
            pl.BlockSpec((_BM, n), lambda i: (i, 0)),
            pl.BlockSpec((n, d), lambda i: (0, 0)),
            pl.BlockSpec((d, h), lambda i: (0, 0)),
            pl.BlockSpec((1, h), lambda i: (0, 0)),
            pl.BlockSpec(memory_space=pltpu.SMEM),
            pl.BlockSpec((h, h), lambda i: (0, 0)),
            pl.BlockSpec((1, h), lambda i: (0, 0)),
            pl.BlockSpec(memory_space=pltpu.SMEM),
        ],
        out_specs=pl.BlockSpec((_BM, h), lambda i: (i, 0)),
        out_shape=jax.ShapeDtypeStruct((n, h), jnp.float32),
        scratch_shapes=[pltpu.VMEM((n, h), jnp.float32)],
        compiler_params=pltpu.CompilerParams(
            dimension_semantics=("arbitrary",),
        ),
    )(A, AX, W0.T, b0.reshape(1, h), a0.reshape(1, 1),
      W1.T, b1.reshape(1, h), a1.reshape(1, 1))

    return out[None, :, :]


# speedup vs baseline: 1.0000x; 1.0000x over previous
import jax
import jax.numpy as jnp
from jax.experimental import pallas as pl
from jax.experimental.pallas import tpu as pltpu

_BM = 400
_NS = 2
_BR = _BM // _NS


def _gcn_kernel(a0_ref, a1_ref_, ax_ref, w0t_ref, b0_ref, a0s_ref, w1t_ref,
                b1_ref, a1s_ref, out_ref, t_ref):
    @pl.when(pl.program_id(0) == 0)
    def _compute_t():
        x = jnp.dot(ax_ref[...], w0t_ref[...], preferred_element_type=jnp.float32)
        x = x + b0_ref[...]
        a0 = a0s_ref[0, 0]
        x = jnp.where(x >= 0, x, a0 * x)
        t = jnp.dot(x, w1t_ref[...], preferred_element_type=jnp.float32)
        t_ref[...] = t + b1_ref[...]

    a1 = a1s_ref[0, 0]
    acc0 = jnp.dot(a0_ref[...], t_ref[...], preferred_element_type=jnp.float32)
    out_ref[0:_BR, :] = jnp.where(acc0 >= 0, acc0, a1 * acc0)
    acc1 = jnp.dot(a1_ref_[...], t_ref[...], preferred_element_type=jnp.float32)
    out_ref[_BR:_BM, :] = jnp.where(acc1 >= 0, acc1, a1 * acc1)


def kernel(A, AX, W0, b0, a0, W1, b1, a1):
    n, d = AX.shape
    h = W0.shape[0]

    out = pl.pallas_call(
        _gcn_kernel,
        grid=(n // _BM,),
        in_specs=[
            pl.BlockSpec((_BR, n), lambda i: (2 * i, 0)),
            pl.BlockSpec((_BR, n), lambda i: (2 * i + 1, 0)),
            pl.BlockSpec((n, d), lambda i: (0, 0)),
            pl.BlockSpec((d, h), lambda i: (0, 0)),
            pl.BlockSpec((1, h), lambda i: (0, 0)),
            pl.BlockSpec(memory_space=pltpu.SMEM),
            pl.BlockSpec((h, h), lambda i: (0, 0)),
            pl.BlockSpec((1, h), lambda i: (0, 0)),
            pl.BlockSpec(memory_space=pltpu.SMEM),
        ],
        out_specs=pl.BlockSpec((_BM, h), lambda i: (i, 0)),
        out_shape=jax.ShapeDtypeStruct((n, h), jnp.float32),
        scratch_shapes=[pltpu.VMEM((n, h), jnp.float32)],
        compiler_params=pltpu.CompilerParams(
            dimension_semantics=("arbitrary",),
        ),
    )(A, A, AX, W0.T, b0.reshape(1, h), a0.reshape(1, 1),
      W1.T, b1.reshape(1, h), a1.reshape(1, 1))

    return out[None, :, :]


# speedup vs baseline: 1.0255x; 1.0255x over previous
import jax
import jax.numpy as jnp
from jax.experimental import pallas as pl
from jax.experimental.pallas import tpu as pltpu

_BM = 400


def _dot_nt(x, w):
    return jax.lax.dot_general(x, w, (((1,), (1,)), ((), ())),
                               preferred_element_type=jnp.float32)


def _gcn_kernel(a_ref, ax_ref, w0_ref, b0_ref, a0_ref, w1_ref, b1_ref,
                a1_ref, out_ref, t_ref):
    @pl.when(pl.program_id(0) == 0)
    def _compute_t():
        x = _dot_nt(ax_ref[...], w0_ref[...]) + b0_ref[...]
        a0 = a0_ref[0]
        x = jnp.where(x >= 0, x, a0 * x)
        t_ref[...] = _dot_nt(x, w1_ref[...]) + b1_ref[...]

    acc = jnp.dot(a_ref[...], t_ref[...], preferred_element_type=jnp.float32)
    a1 = a1_ref[0]
    out_ref[...] = jnp.where(acc >= 0, acc, a1 * acc)


def kernel(A, AX, W0, b0, a0, W1, b1, a1):
    n, d = AX.shape
    h = W0.shape[0]

    out = pl.pallas_call(
        _gcn_kernel,
        grid=(n // _BM,),
        in_specs=[
            pl.BlockSpec((_BM, n), lambda i: (i, 0)),
            pl.BlockSpec((n, d), lambda i: (0, 0)),
            pl.BlockSpec((h, d), lambda i: (0, 0)),
            pl.BlockSpec((1, h), lambda i: (0, 0)),
            pl.BlockSpec(memory_space=pltpu.SMEM),
            pl.BlockSpec((h, h), lambda i: (0, 0)),
            pl.BlockSpec((1, h), lambda i: (0, 0)),
            pl.BlockSpec(memory_space=pltpu.SMEM),
        ],
        out_specs=pl.BlockSpec((_BM, h), lambda i: (i, 0)),
        out_shape=jax.ShapeDtypeStruct((n, h), jnp.float32),
        scratch_shapes=[pltpu.VMEM((n, h), jnp.float32)],
        compiler_params=pltpu.CompilerParams(
            dimension_semantics=("arbitrary",),
        ),
    )(A, AX, W0, b0.reshape(1, h), a0.reshape(1), W1, b1.reshape(1, h),
      a1.reshape(1))

    return out[None, :, :]
